# baseline (speedup 1.0000x reference)
import functools

import jax
import jax.numpy as jnp
from jax import lax
from jax.experimental import pallas as pl
from jax.experimental.pallas import tpu as pltpu
from jax.experimental.pallas import tpu_sc as plsc

N = 10000
E = 320000
F_N = 128
F_E = 4
H = 128

NC = 2
NS = 16
NW = NC * NS
BE = 128
NBLK = E // BE
EU = 4
HC = H // 16

SW = 8
N_PAD = 10240

NFULL = NBLK // NW
NREM = NBLK % NW
NPAIR = NFULL // 2


def _tables_body(x_ref, wst_ref, wdt_ref, be1_ref, p_ref, q_ref):
    xv = x_ref[...]
    p_ref[...] = (
        jnp.dot(xv, wst_ref[...], preferred_element_type=jnp.float32)
        + be1_ref[...]
    )
    q_ref[...] = jnp.dot(xv, wdt_ref[...], preferred_element_type=jnp.float32)


def _node_tables(x, wst, wdt, be1):
    blk = 2000
    grid = N // blk
    return pl.pallas_call(
        _tables_body,
        grid=(grid,),
        in_specs=[
            pl.BlockSpec((blk, F_N), lambda i: (i, 0)),
            pl.BlockSpec((F_N, H), lambda i: (0, 0)),
            pl.BlockSpec((F_N, H), lambda i: (0, 0)),
            pl.BlockSpec((1, H), lambda i: (0, 0)),
        ],
        out_specs=[
            pl.BlockSpec((blk, H), lambda i: (i, 0)),
            pl.BlockSpec((blk, H), lambda i: (i, 0)),
        ],
        out_shape=[
            jax.ShapeDtypeStruct((N, H), jnp.float32),
            jax.ShapeDtypeStruct((N, H), jnp.float32),
        ],
    )(x, wst, wdt, be1)


EBLK = 2560


def _attr_t_body(a_ref, out_ref):
    at = a_ref[...].T
    out_ref[...] = jnp.concatenate(
        [at, jnp.zeros((F_E, EBLK), jnp.float32)], axis=0)


def _attr_t(attr):
    return pl.pallas_call(
        _attr_t_body,
        grid=(E // EBLK,),
        in_specs=[pl.BlockSpec((EBLK, F_E), lambda i: (i, 0))],
        out_specs=pl.BlockSpec((2 * F_E, EBLK), lambda i: (0, i)),
        out_shape=jax.ShapeDtypeStruct((2 * F_E, E), jnp.float32),
    )(attr)


def _emb_t_body(e_ref, out_ref):
    out_ref[...] = e_ref[0:F_E, :].T


def _emb_t(embt):
    return pl.pallas_call(
        _emb_t_body,
        grid=(E // EBLK,),
        in_specs=[pl.BlockSpec((2 * F_E, EBLK), lambda i: (0, i))],
        out_specs=pl.BlockSpec((EBLK, F_E), lambda i: (i, 0)),
        out_shape=jax.ShapeDtypeStruct((E, F_E), jnp.float32),
    )(embt)


def _sc_edge_kernel(p_hbm, q_hbm, row_hbm, col_hbm, attr_hbm, wat_hbm,
                    we2_hbm, be2_hbm, zeros8_hbm,
                    emb_hbm, seg_hbm,
                    row0_v, col0_v, attr0_v, row1_v, col1_v, attr1_v,
                    p0_v, q0_v, p1_v, q1_v, emb_v, scat_v,
                    wat_v, we2_v, be2_v, zbuf_v, seg_sh,
                    sem_i0, sem_i1, sem_p0, sem_q0, sem_p1, sem_q1):
    cid = lax.axis_index("c")
    sid = lax.axis_index("s")
    wid = cid * NS + sid

    pltpu.sync_copy(wat_hbm, wat_v)
    pltpu.sync_copy(we2_hbm, we2_v)
    pltpu.sync_copy(be2_hbm, be2_v)

    iota = lax.iota(jnp.int32, 16)
    ediv = iota // F_E
    jmod = iota % F_E
    rdiv = iota // SW
    rmod = iota % SW

    cnt_pat = jnp.where(rmod == F_E, 1.0, 0.0)
    for k in range(BE * SW // 16):
        plsc.store_scatter(scat_v, (k * 2 + rdiv, rmod), cnt_pat)

    off = sid * (N_PAD // NS)
    pltpu.sync_copy(zeros8_hbm.at[pl.ds(off, N_PAD // NS)], zbuf_v)
    pltpu.sync_copy(zbuf_v, seg_sh.at[pl.ds(off, N_PAD // NS)])

    plsc.subcore_barrier()

    be2_tile = be2_v[pl.ds(0, 16)]

    def blk_base(b):
        return (wid + b * NW) * BE

    def issue_idx(b, rv, cv, av, sem):
        base = blk_base(b)
        pltpu.async_copy(row_hbm.at[pl.ds(base, BE)], rv, sem)
        pltpu.async_copy(col_hbm.at[pl.ds(base, BE)], cv, sem)
        pltpu.async_copy(attr_hbm.at[:, pl.ds(base, BE)], av, sem)

    def wait_idx(b, rv, cv, av, sem):
        base = blk_base(b)
        pltpu.make_async_copy(row_hbm.at[pl.ds(base, BE)], rv, sem).wait()
        pltpu.make_async_copy(col_hbm.at[pl.ds(base, BE)], cv, sem).wait()
        pltpu.make_async_copy(attr_hbm.at[:, pl.ds(base, BE)], av,
                              sem).wait()

    def issue_gathers(rv, cv, pv, qv, semp, semq):
        pltpu.async_copy(p_hbm.at[rv], pv, semp)
        pltpu.async_copy(q_hbm.at[cv], qv, semq)

    def wait_gathers(rv, cv, pv, qv, semp, semq):
        pltpu.make_async_copy(p_hbm.at[rv], pv, semp).wait()
        pltpu.make_async_copy(q_hbm.at[cv], qv, semq).wait()

    def compute_and_out(b, rv, av, pv, qv):
        def group_body(g, gcarry):
            avec = plsc.load_gather(av, (jmod, g * EU + ediv))
            out_vec = be2_tile
            for u in range(EU):
                e = g * EU + u
                accs = [None] * F_E
                for c in range(HC):
                    sl = pl.ds(c * 16, 16)
                    pre = pv[e, sl] + qv[e, sl]
                    pre = pre + avec[EU * u + 0] * wat_v[0, sl]
                    pre = pre + avec[EU * u + 1] * wat_v[1, sl]
                    pre = pre + avec[EU * u + 2] * wat_v[2, sl]
                    pre = pre + avec[EU * u + 3] * wat_v[3, sl]
                    h = jnp.maximum(pre, 0.0)
                    for j in range(F_E):
                        t = h * we2_v[j, sl]
                        accs[j] = t if c == 0 else accs[j] + t
                for j in range(F_E):
                    out_vec = jnp.where(iota == F_E * u + j,
                                        jnp.sum(accs[j]) + out_vec, out_vec)
            plsc.store_scatter(emb_v, (jmod, g * EU + ediv), out_vec)
            plsc.store_scatter(scat_v, (g * EU + ediv, jmod), out_vec)
            return gcarry

        lax.fori_loop(0, BE // EU, group_body, 0)
        pltpu.sync_copy(emb_v, emb_hbm.at[:, pl.ds(blk_base(b), BE)])
        pltpu.sync_copy(scat_v, seg_sh.at[rv], add=True)

    issue_idx(0, row0_v, col0_v, attr0_v, sem_i0)
    wait_idx(0, row0_v, col0_v, attr0_v, sem_i0)
    issue_gathers(row0_v, col0_v, p0_v, q0_v, sem_p0, sem_q0)
    issue_idx(1, row1_v, col1_v, attr1_v, sem_i1)

    def pair_body(t, carry):
        b0 = 2 * t
        b1 = 2 * t + 1
        wait_idx(b1, row1_v, col1_v, attr1_v, sem_i1)
        issue_gathers(row1_v, col1_v, p1_v, q1_v, sem_p1, sem_q1)
        wait_gathers(row0_v, col0_v, p0_v, q0_v, sem_p0, sem_q0)
        compute_and_out(b0, row0_v, attr0_v, p0_v, q0_v)

        @pl.when(t < NPAIR - 1)
        def _():
            issue_idx(b0 + 2, row0_v, col0_v, attr0_v, sem_i0)
            wait_idx(b0 + 2, row0_v, col0_v, attr0_v, sem_i0)
            issue_gathers(row0_v, col0_v, p0_v, q0_v, sem_p0, sem_q0)

        wait_gathers(row1_v, col1_v, p1_v, q1_v, sem_p1, sem_q1)
        compute_and_out(b1, row1_v, attr1_v, p1_v, q1_v)

        @pl.when(t < NPAIR - 1)
        def _():
            issue_idx(b1 + 2, row1_v, col1_v, attr1_v, sem_i1)

        return carry

    lax.fori_loop(0, NPAIR, pair_body, 0)

    @pl.when(wid < NREM)
    def _():
        issue_idx(NFULL, row0_v, col0_v, attr0_v, sem_i0)
        wait_idx(NFULL, row0_v, col0_v, attr0_v, sem_i0)
        issue_gathers(row0_v, col0_v, p0_v, q0_v, sem_p0, sem_q0)
        wait_gathers(row0_v, col0_v, p0_v, q0_v, sem_p0, sem_q0)
        compute_and_out(NFULL, row0_v, attr0_v, p0_v, q0_v)

    plsc.subcore_barrier()

    pltpu.sync_copy(seg_sh.at[pl.ds(off, N_PAD // NS)], zbuf_v)
    pltpu.sync_copy(zbuf_v, seg_hbm.at[cid, pl.ds(off, N_PAD // NS)])


def _sc_edge_pass(p, q, row, col, attr_flat, wat, we2, be2_tiled):
    mesh = plsc.VectorSubcoreMesh(core_axis_name="c", subcore_axis_name="s",
                                  num_cores=NC, num_subcores=NS)
    zeros8 = jnp.zeros((N_PAD, SW), jnp.float32)
    fn = functools.partial(
        pl.kernel,
        out_type=(
            jax.ShapeDtypeStruct((2 * F_E, E), jnp.float32),
            jax.ShapeDtypeStruct((NC, N_PAD, SW), jnp.float32),
        ),
        mesh=mesh,
        compiler_params=pltpu.CompilerParams(needs_layout_passes=False,
                                             use_tc_tiling_on_sc=False),
        scratch_types=[
            pltpu.VMEM((BE,), jnp.int32),
            pltpu.VMEM((BE,), jnp.int32),
            pltpu.VMEM((2 * F_E, BE), jnp.float32),
            pltpu.VMEM((BE,), jnp.int32),
            pltpu.VMEM((BE,), jnp.int32),
            pltpu.VMEM((2 * F_E, BE), jnp.float32),
            pltpu.VMEM((BE, H), jnp.float32),
            pltpu.VMEM((BE, H), jnp.float32),
            pltpu.VMEM((BE, H), jnp.float32),
            pltpu.VMEM((BE, H), jnp.float32),
            pltpu.VMEM((2 * F_E, BE), jnp.float32),
            pltpu.VMEM((BE, SW), jnp.float32),
            pltpu.VMEM((F_E, H), jnp.float32),
            pltpu.VMEM((F_E, H), jnp.float32),
            pltpu.VMEM((16,), jnp.float32),
            pltpu.VMEM((N_PAD // NS, SW), jnp.float32),
            pltpu.VMEM_SHARED((N_PAD, SW), jnp.float32),
            pltpu.SemaphoreType.DMA,
            pltpu.SemaphoreType.DMA,
            pltpu.SemaphoreType.DMA,
            pltpu.SemaphoreType.DMA,
            pltpu.SemaphoreType.DMA,
            pltpu.SemaphoreType.DMA,
        ],
    )(_sc_edge_kernel)
    return fn(p, q, row, col, attr_flat, wat, we2, be2_tiled, zeros8)


def _node_mlp_body(x_ref, seg_ref, wn1xt_ref, wn1at_ref, bn1_ref,
                   wn2t_ref, bn2_ref, out_ref):
    tot = seg_ref[0] + seg_ref[1]
    seg = tot[:, 0:F_E]
    cnt = tot[:, F_E:F_E + 1]
    agg = seg / jnp.maximum(cnt, 1.0)
    h2 = (
        jnp.dot(x_ref[...], wn1xt_ref[...], preferred_element_type=jnp.float32)
        + jnp.dot(agg, wn1at_ref[...], preferred_element_type=jnp.float32)
        + bn1_ref[...]
    )
    h2 = jnp.maximum(h2, 0.0)
    out_ref[...] = (
        jnp.dot(h2, wn2t_ref[...], preferred_element_type=jnp.float32)
        + bn2_ref[...]
    )


def _node_mlp(x, seg, wn1xt, wn1at, bn1, wn2t, bn2):
    blk = 2000
    grid = N // blk
    return pl.pallas_call(
        _node_mlp_body,
        grid=(grid,),
        in_specs=[
            pl.BlockSpec((blk, F_N), lambda i: (i, 0)),
            pl.BlockSpec((NC, blk, SW), lambda i: (0, i, 0)),
            pl.BlockSpec((F_N, H), lambda i: (0, 0)),
            pl.BlockSpec((F_E, H), lambda i: (0, 0)),
            pl.BlockSpec((1, H), lambda i: (0, 0)),
            pl.BlockSpec((H, F_N), lambda i: (0, 0)),
            pl.BlockSpec((1, F_N), lambda i: (0, 0)),
        ],
        out_specs=pl.BlockSpec((blk, F_N), lambda i: (i, 0)),
        out_shape=jax.ShapeDtypeStruct((N, F_N), jnp.float32),
    )(x, seg, wn1xt, wn1at, bn1, wn2t, bn2)


def kernel(x, edge_index, edge_attr, We1, be1, We2, be2, Wn1, bn1, Wn2, bn2):
    row = edge_index[0].astype(jnp.int32)
    col = edge_index[1].astype(jnp.int32)

    wst = We1[:, :F_N].T
    wdt = We1[:, F_N:2 * F_N].T
    wat = We1[:, 2 * F_N:].T
    be1_2d = be1.reshape(1, H)
    be2_tiled = jnp.tile(be2, 16 // F_E)
    wn1xt = Wn1[:, :F_N].T
    wn1at = Wn1[:, F_N:].T
    bn1_2d = bn1.reshape(1, H)
    wn2t = Wn2.T
    bn2_2d = bn2.reshape(1, F_N)

    p, q = _node_tables(x, wst, wdt, be1_2d)
    attr_t = _attr_t(edge_attr)
    embt, seg = _sc_edge_pass(p, q, row, col, attr_t, wat, We2, be2_tiled)
    emb = _emb_t(embt)
    node_embeddings = _node_mlp(x, seg, wn1xt, wn1at, bn1_2d, wn2t, bn2_2d)
    return (emb, node_embeddings)


# speedup vs baseline: 1.2825x; 1.2825x over previous
import functools

import jax
import jax.numpy as jnp
from jax import lax
from jax.experimental import pallas as pl
from jax.experimental.pallas import tpu as pltpu
from jax.experimental.pallas import tpu_sc as plsc

N = 10000
E = 320000
F_N = 128
F_E = 4
H = 128

NC = 2
NS = 16
NW = NC * NS
BE = 128
NBLK = E // BE
EU = 4
HC = H // 16

SW = 8
N_PAD = 10240

NFULL = NBLK // NW
NREM = NBLK % NW
NPAIR = NFULL // 2


def _tables_body(x_ref, wst_ref, wdt_ref, be1_ref, p_ref, q_ref):
    xv = x_ref[...]
    p_ref[...] = (
        jnp.dot(xv, wst_ref[...], preferred_element_type=jnp.float32)
        + be1_ref[...]
    )
    q_ref[...] = jnp.dot(xv, wdt_ref[...], preferred_element_type=jnp.float32)


def _node_tables(x, wst, wdt, be1):
    blk = 2000
    grid = N // blk
    return pl.pallas_call(
        _tables_body,
        grid=(grid,),
        in_specs=[
            pl.BlockSpec((blk, F_N), lambda i: (i, 0)),
            pl.BlockSpec((F_N, H), lambda i: (0, 0)),
            pl.BlockSpec((F_N, H), lambda i: (0, 0)),
            pl.BlockSpec((1, H), lambda i: (0, 0)),
        ],
        out_specs=[
            pl.BlockSpec((blk, H), lambda i: (i, 0)),
            pl.BlockSpec((blk, H), lambda i: (i, 0)),
        ],
        out_shape=[
            jax.ShapeDtypeStruct((N, H), jnp.float32),
            jax.ShapeDtypeStruct((N, H), jnp.float32),
        ],
    )(x, wst, wdt, be1)


EBLK = 12800


def _a128_body(a_ref, wat_ref, out_ref):
    out_ref[...] = jnp.dot(a_ref[...], wat_ref[...],
                           preferred_element_type=jnp.float32)


def _a128(attr, wat):
    return pl.pallas_call(
        _a128_body,
        grid=(E // EBLK,),
        in_specs=[
            pl.BlockSpec((EBLK, F_E), lambda i: (i, 0)),
            pl.BlockSpec((F_E, H), lambda i: (0, 0)),
        ],
        out_specs=pl.BlockSpec((EBLK, H), lambda i: (i, 0)),
        out_shape=jax.ShapeDtypeStruct((E, H), jnp.float32),
    )(attr, wat)


def _sc_edge_kernel(p_hbm, q_hbm, row_hbm, col_hbm, a_hbm,
                    we2_hbm, be2_hbm, zeros8_hbm,
                    emb_hbm, seg_hbm,
                    row0_v, col0_v, a0_v, row1_v, col1_v, a1_v,
                    p0_v, q0_v, p1_v, q1_v, emb_v, scat_v,
                    we2_v, be2_v, zbuf_v, seg_sh,
                    sem_i0, sem_i1, sem_p0, sem_q0, sem_p1, sem_q1):
    cid = lax.axis_index("c")
    sid = lax.axis_index("s")
    wid = cid * NS + sid

    pltpu.sync_copy(we2_hbm, we2_v)
    pltpu.sync_copy(be2_hbm, be2_v)

    iota = lax.iota(jnp.int32, 16)
    ediv = iota // F_E
    jmod = iota % F_E
    rdiv = iota // SW
    rmod = iota % SW

    cnt_pat = jnp.where(rmod == F_E, 1.0, 0.0)
    for k in range(BE * SW // 16):
        plsc.store_scatter(scat_v, (k * 2 + rdiv, rmod), cnt_pat)

    off = sid * (N_PAD // NS)
    pltpu.sync_copy(zeros8_hbm.at[pl.ds(off, N_PAD // NS)], zbuf_v)
    pltpu.sync_copy(zbuf_v, seg_sh.at[pl.ds(off, N_PAD // NS)])

    plsc.subcore_barrier()

    be2_tile = be2_v[pl.ds(0, 16)]

    def blk_base(b):
        return (wid + b * NW) * BE

    def issue_idx(b, rv, cv, av, sem):
        base = blk_base(b)
        pltpu.async_copy(row_hbm.at[pl.ds(base, BE)], rv, sem)
        pltpu.async_copy(col_hbm.at[pl.ds(base, BE)], cv, sem)
        pltpu.async_copy(a_hbm.at[pl.ds(base, BE)], av, sem)

    def wait_idx(b, rv, cv, av, sem):
        base = blk_base(b)
        pltpu.make_async_copy(row_hbm.at[pl.ds(base, BE)], rv, sem).wait()
        pltpu.make_async_copy(col_hbm.at[pl.ds(base, BE)], cv, sem).wait()
        pltpu.make_async_copy(a_hbm.at[pl.ds(base, BE)], av, sem).wait()

    def issue_gathers(rv, cv, pv, qv, semp, semq):
        pltpu.async_copy(p_hbm.at[rv], pv, semp)
        pltpu.async_copy(q_hbm.at[cv], qv, semq)

    def wait_gathers(rv, cv, pv, qv, semp, semq):
        pltpu.make_async_copy(p_hbm.at[rv], pv, semp).wait()
        pltpu.make_async_copy(q_hbm.at[cv], qv, semq).wait()

    def compute_and_out(b, rv, av, pv, qv):
        def group_body(g, gcarry):
            out_vec = be2_tile
            for u in range(EU):
                e = g * EU + u
                accs = [None] * F_E
                for c in range(HC):
                    sl = pl.ds(c * 16, 16)
                    pre = pv[e, sl] + qv[e, sl] + av[e, sl]
                    h = jnp.maximum(pre, 0.0)
                    for j in range(F_E):
                        t = h * we2_v[j, sl]
                        accs[j] = t if c == 0 else accs[j] + t
                for j in range(F_E):
                    out_vec = jnp.where(iota == F_E * u + j,
                                        jnp.sum(accs[j]) + out_vec, out_vec)
            plsc.store_scatter(emb_v, (g * EU + ediv, jmod), out_vec)
            plsc.store_scatter(scat_v, (g * EU + ediv, jmod), out_vec)
            return gcarry

        lax.fori_loop(0, BE // EU, group_body, 0)
        pltpu.sync_copy(emb_v, emb_hbm.at[pl.ds(blk_base(b), BE)])
        pltpu.sync_copy(scat_v, seg_sh.at[rv], add=True)

    issue_idx(0, row0_v, col0_v, a0_v, sem_i0)
    wait_idx(0, row0_v, col0_v, a0_v, sem_i0)
    issue_gathers(row0_v, col0_v, p0_v, q0_v, sem_p0, sem_q0)
    issue_idx(1, row1_v, col1_v, a1_v, sem_i1)

    def pair_body(t, carry):
        b0 = 2 * t
        b1 = 2 * t + 1
        wait_idx(b1, row1_v, col1_v, a1_v, sem_i1)
        issue_gathers(row1_v, col1_v, p1_v, q1_v, sem_p1, sem_q1)
        wait_gathers(row0_v, col0_v, p0_v, q0_v, sem_p0, sem_q0)
        compute_and_out(b0, row0_v, a0_v, p0_v, q0_v)

        @pl.when(t < NPAIR - 1)
        def _():
            issue_idx(b0 + 2, row0_v, col0_v, a0_v, sem_i0)
            wait_idx(b0 + 2, row0_v, col0_v, a0_v, sem_i0)
            issue_gathers(row0_v, col0_v, p0_v, q0_v, sem_p0, sem_q0)

        wait_gathers(row1_v, col1_v, p1_v, q1_v, sem_p1, sem_q1)
        compute_and_out(b1, row1_v, a1_v, p1_v, q1_v)

        @pl.when(t < NPAIR - 1)
        def _():
            issue_idx(b1 + 2, row1_v, col1_v, a1_v, sem_i1)

        return carry

    lax.fori_loop(0, NPAIR, pair_body, 0)

    @pl.when(wid < NREM)
    def _():
        issue_idx(NFULL, row0_v, col0_v, a0_v, sem_i0)
        wait_idx(NFULL, row0_v, col0_v, a0_v, sem_i0)
        issue_gathers(row0_v, col0_v, p0_v, q0_v, sem_p0, sem_q0)
        wait_gathers(row0_v, col0_v, p0_v, q0_v, sem_p0, sem_q0)
        compute_and_out(NFULL, row0_v, a0_v, p0_v, q0_v)

    plsc.subcore_barrier()

    pltpu.sync_copy(seg_sh.at[pl.ds(off, N_PAD // NS)], zbuf_v)
    pltpu.sync_copy(zbuf_v, seg_hbm.at[cid, pl.ds(off, N_PAD // NS)])


def _sc_edge_pass(p, q, row, col, a128, we2, be2_tiled):
    mesh = plsc.VectorSubcoreMesh(core_axis_name="c", subcore_axis_name="s",
                                  num_cores=NC, num_subcores=NS)
    zeros8 = jnp.zeros((N_PAD, SW), jnp.float32)
    fn = functools.partial(
        pl.kernel,
        out_type=(
            jax.ShapeDtypeStruct((E, F_E), jnp.float32),
            jax.ShapeDtypeStruct((NC, N_PAD, SW), jnp.float32),
        ),
        mesh=mesh,
        compiler_params=pltpu.CompilerParams(needs_layout_passes=False,
                                             use_tc_tiling_on_sc=False),
        scratch_types=[
            pltpu.VMEM((BE,), jnp.int32),
            pltpu.VMEM((BE,), jnp.int32),
            pltpu.VMEM((BE, H), jnp.float32),
            pltpu.VMEM((BE,), jnp.int32),
            pltpu.VMEM((BE,), jnp.int32),
            pltpu.VMEM((BE, H), jnp.float32),
            pltpu.VMEM((BE, H), jnp.float32),
            pltpu.VMEM((BE, H), jnp.float32),
            pltpu.VMEM((BE, H), jnp.float32),
            pltpu.VMEM((BE, H), jnp.float32),
            pltpu.VMEM((BE, F_E), jnp.float32),
            pltpu.VMEM((BE, SW), jnp.float32),
            pltpu.VMEM((F_E, H), jnp.float32),
            pltpu.VMEM((16,), jnp.float32),
            pltpu.VMEM((N_PAD // NS, SW), jnp.float32),
            pltpu.VMEM_SHARED((N_PAD, SW), jnp.float32),
            pltpu.SemaphoreType.DMA,
            pltpu.SemaphoreType.DMA,
            pltpu.SemaphoreType.DMA,
            pltpu.SemaphoreType.DMA,
            pltpu.SemaphoreType.DMA,
            pltpu.SemaphoreType.DMA,
        ],
    )(_sc_edge_kernel)
    return fn(p, q, row, col, a128, we2, be2_tiled, zeros8)


def _node_mlp_body(x_ref, seg_ref, wn1xt_ref, wn1at_ref, bn1_ref,
                   wn2t_ref, bn2_ref, out_ref):
    tot = seg_ref[0] + seg_ref[1]
    seg = tot[:, 0:F_E]
    cnt = tot[:, F_E:F_E + 1]
    agg = seg / jnp.maximum(cnt, 1.0)
    h2 = (
        jnp.dot(x_ref[...], wn1xt_ref[...], preferred_element_type=jnp.float32)
        + jnp.dot(agg, wn1at_ref[...], preferred_element_type=jnp.float32)
        + bn1_ref[...]
    )
    h2 = jnp.maximum(h2, 0.0)
    out_ref[...] = (
        jnp.dot(h2, wn2t_ref[...], preferred_element_type=jnp.float32)
        + bn2_ref[...]
    )


def _node_mlp(x, seg, wn1xt, wn1at, bn1, wn2t, bn2):
    blk = 2000
    grid = N // blk
    return pl.pallas_call(
        _node_mlp_body,
        grid=(grid,),
        in_specs=[
            pl.BlockSpec((blk, F_N), lambda i: (i, 0)),
            pl.BlockSpec((NC, blk, SW), lambda i: (0, i, 0)),
            pl.BlockSpec((F_N, H), lambda i: (0, 0)),
            pl.BlockSpec((F_E, H), lambda i: (0, 0)),
            pl.BlockSpec((1, H), lambda i: (0, 0)),
            pl.BlockSpec((H, F_N), lambda i: (0, 0)),
            pl.BlockSpec((1, F_N), lambda i: (0, 0)),
        ],
        out_specs=pl.BlockSpec((blk, F_N), lambda i: (i, 0)),
        out_shape=jax.ShapeDtypeStruct((N, F_N), jnp.float32),
    )(x, seg, wn1xt, wn1at, bn1, wn2t, bn2)


def kernel(x, edge_index, edge_attr, We1, be1, We2, be2, Wn1, bn1, Wn2, bn2):
    row = edge_index[0].astype(jnp.int32)
    col = edge_index[1].astype(jnp.int32)

    wst = We1[:, :F_N].T
    wdt = We1[:, F_N:2 * F_N].T
    wat = We1[:, 2 * F_N:].T
    be1_2d = be1.reshape(1, H)
    be2_tiled = jnp.tile(be2, 16 // F_E)
    wn1xt = Wn1[:, :F_N].T
    wn1at = Wn1[:, F_N:].T
    bn1_2d = bn1.reshape(1, H)
    wn2t = Wn2.T
    bn2_2d = bn2.reshape(1, F_N)

    p, q = _node_tables(x, wst, wdt, be1_2d)
    a128 = _a128(edge_attr, wat)
    emb, seg = _sc_edge_pass(p, q, row, col, a128, We2, be2_tiled)
    node_embeddings = _node_mlp(x, seg, wn1xt, wn1at, bn1_2d, wn2t, bn2_2d)
    return (emb, node_embeddings)


# speedup vs baseline: 1.2946x; 1.0094x over previous
import functools

import jax
import jax.numpy as jnp
from jax import lax
from jax.experimental import pallas as pl
from jax.experimental.pallas import tpu as pltpu
from jax.experimental.pallas import tpu_sc as plsc

N = 10000
E = 320000
F_N = 128
F_E = 4
H = 128

NC = 2
NS = 16
NW = NC * NS
BE = 128
NBLK = E // BE
EU = 4
HC = H // 16

SW = 8
N_PAD = 10240

NFULL = NBLK // NW
NREM = NBLK % NW
NPAIR = NFULL // 2


def _tables_body(x_ref, wst_ref, wdt_ref, be1_ref, p_ref, q_ref):
    xv = x_ref[...]
    p_ref[...] = (
        jnp.dot(xv, wst_ref[...], preferred_element_type=jnp.float32)
        + be1_ref[...]
    )
    q_ref[...] = jnp.dot(xv, wdt_ref[...], preferred_element_type=jnp.float32)


def _node_tables(x, wst, wdt, be1):
    blk = 2000
    grid = N // blk
    return pl.pallas_call(
        _tables_body,
        grid=(grid,),
        in_specs=[
            pl.BlockSpec((blk, F_N), lambda i: (i, 0)),
            pl.BlockSpec((F_N, H), lambda i: (0, 0)),
            pl.BlockSpec((F_N, H), lambda i: (0, 0)),
            pl.BlockSpec((1, H), lambda i: (0, 0)),
        ],
        out_specs=[
            pl.BlockSpec((blk, H), lambda i: (i, 0)),
            pl.BlockSpec((blk, H), lambda i: (i, 0)),
        ],
        out_shape=[
            jax.ShapeDtypeStruct((N, H), jnp.float32),
            jax.ShapeDtypeStruct((N, H), jnp.float32),
        ],
    )(x, wst, wdt, be1)


EBLK = 12800


def _a128_body(a_ref, wat_ref, out_ref):
    out_ref[...] = jnp.dot(a_ref[...], wat_ref[...],
                           preferred_element_type=jnp.float32)


def _a128(attr, wat):
    return pl.pallas_call(
        _a128_body,
        grid=(E // EBLK,),
        in_specs=[
            pl.BlockSpec((EBLK, F_E), lambda i: (i, 0)),
            pl.BlockSpec((F_E, H), lambda i: (0, 0)),
        ],
        out_specs=pl.BlockSpec((EBLK, H), lambda i: (i, 0)),
        out_shape=jax.ShapeDtypeStruct((E, H), jnp.float32),
    )(attr, wat)


def _sc_edge_kernel(p_hbm, q_hbm, row_hbm, col_hbm, a_hbm,
                    we2_hbm, be2_hbm, zeros8_hbm,
                    emb_hbm, seg_hbm,
                    row0_v, col0_v, a0_v, row1_v, col1_v, a1_v,
                    p0_v, q0_v, p1_v, q1_v, emb0_v, emb1_v, scat_v,
                    we2_v, be2_v, zbuf_v, seg_sh,
                    sem_i0, sem_i1, sem_p0, sem_q0, sem_p1, sem_q1,
                    sem_o0, sem_o1):
    cid = lax.axis_index("c")
    sid = lax.axis_index("s")
    wid = cid * NS + sid

    pltpu.sync_copy(we2_hbm, we2_v)
    pltpu.sync_copy(be2_hbm, be2_v)

    iota = lax.iota(jnp.int32, 16)
    ediv = iota // F_E
    jmod = iota % F_E
    rdiv = iota // SW
    rmod = iota % SW

    cnt_pat = jnp.where(rmod == F_E, 1.0, 0.0)
    for k in range(BE * SW // 16):
        plsc.store_scatter(scat_v, (k * 2 + rdiv, rmod), cnt_pat)

    off = sid * (N_PAD // NS)
    pltpu.sync_copy(zeros8_hbm.at[pl.ds(off, N_PAD // NS)], zbuf_v)
    pltpu.sync_copy(zbuf_v, seg_sh.at[pl.ds(off, N_PAD // NS)])

    plsc.subcore_barrier()

    be2_tile = be2_v[pl.ds(0, 16)]

    def blk_base(b):
        return (wid + b * NW) * BE

    def issue_idx(b, rv, cv, av, sem):
        base = blk_base(b)
        pltpu.async_copy(row_hbm.at[pl.ds(base, BE)], rv, sem)
        pltpu.async_copy(col_hbm.at[pl.ds(base, BE)], cv, sem)
        pltpu.async_copy(a_hbm.at[pl.ds(base, BE)], av, sem)

    def wait_idx(b, rv, cv, av, sem):
        base = blk_base(b)
        pltpu.make_async_copy(row_hbm.at[pl.ds(base, BE)], rv, sem).wait()
        pltpu.make_async_copy(col_hbm.at[pl.ds(base, BE)], cv, sem).wait()
        pltpu.make_async_copy(a_hbm.at[pl.ds(base, BE)], av, sem).wait()

    def issue_gathers(rv, cv, pv, qv, semp, semq):
        pltpu.async_copy(p_hbm.at[rv], pv, semp)
        pltpu.async_copy(q_hbm.at[cv], qv, semq)

    def wait_gathers(rv, cv, pv, qv, semp, semq):
        pltpu.make_async_copy(p_hbm.at[rv], pv, semp).wait()
        pltpu.make_async_copy(q_hbm.at[cv], qv, semq).wait()

    def compute_and_out(b, rv, av, pv, qv, ev, sem_o, sync_out):
        def group_body(g, gcarry):
            out_vec = be2_tile
            for u in range(EU):
                e = g * EU + u
                accs = [None] * F_E
                for c in range(HC):
                    sl = pl.ds(c * 16, 16)
                    pre = pv[e, sl] + qv[e, sl] + av[e, sl]
                    h = jnp.maximum(pre, 0.0)
                    for j in range(F_E):
                        t = h * we2_v[j, sl]
                        accs[j] = t if c == 0 else accs[j] + t
                for j in range(F_E):
                    out_vec = jnp.where(iota == F_E * u + j,
                                        jnp.sum(accs[j]) + out_vec, out_vec)
            plsc.store_scatter(ev, (g * EU + ediv, jmod), out_vec)
            plsc.store_scatter(scat_v, (g * EU + ediv, jmod), out_vec)
            return gcarry

        lax.fori_loop(0, BE // EU, group_body, 0)
        if sync_out:
            pltpu.sync_copy(ev, emb_hbm.at[pl.ds(blk_base(b), BE)])
        else:
            pltpu.async_copy(ev, emb_hbm.at[pl.ds(blk_base(b), BE)], sem_o)
        pltpu.sync_copy(scat_v, seg_sh.at[rv], add=True)

    issue_idx(0, row0_v, col0_v, a0_v, sem_i0)
    wait_idx(0, row0_v, col0_v, a0_v, sem_i0)
    issue_gathers(row0_v, col0_v, p0_v, q0_v, sem_p0, sem_q0)
    issue_idx(1, row1_v, col1_v, a1_v, sem_i1)

    def pair_body(t, carry):
        b0 = 2 * t
        b1 = 2 * t + 1
        wait_idx(b1, row1_v, col1_v, a1_v, sem_i1)
        issue_gathers(row1_v, col1_v, p1_v, q1_v, sem_p1, sem_q1)
        wait_gathers(row0_v, col0_v, p0_v, q0_v, sem_p0, sem_q0)

        @pl.when(t > 0)
        def _():
            pltpu.make_async_copy(
                emb0_v, emb_hbm.at[pl.ds(blk_base(b0 - 2), BE)],
                sem_o0).wait()

        compute_and_out(b0, row0_v, a0_v, p0_v, q0_v, emb0_v, sem_o0, False)

        @pl.when(t < NPAIR - 1)
        def _():
            issue_idx(b0 + 2, row0_v, col0_v, a0_v, sem_i0)
            wait_idx(b0 + 2, row0_v, col0_v, a0_v, sem_i0)
            issue_gathers(row0_v, col0_v, p0_v, q0_v, sem_p0, sem_q0)

        wait_gathers(row1_v, col1_v, p1_v, q1_v, sem_p1, sem_q1)

        @pl.when(t > 0)
        def _():
            pltpu.make_async_copy(
                emb1_v, emb_hbm.at[pl.ds(blk_base(b1 - 2), BE)],
                sem_o1).wait()

        compute_and_out(b1, row1_v, a1_v, p1_v, q1_v, emb1_v, sem_o1, False)

        @pl.when(t < NPAIR - 1)
        def _():
            issue_idx(b1 + 2, row1_v, col1_v, a1_v, sem_i1)

        return carry

    lax.fori_loop(0, NPAIR, pair_body, 0)

    pltpu.make_async_copy(emb0_v, emb_hbm.at[pl.ds(blk_base(NFULL - 2), BE)],
                          sem_o0).wait()
    pltpu.make_async_copy(emb1_v, emb_hbm.at[pl.ds(blk_base(NFULL - 1), BE)],
                          sem_o1).wait()

    @pl.when(wid < NREM)
    def _():
        issue_idx(NFULL, row0_v, col0_v, a0_v, sem_i0)
        wait_idx(NFULL, row0_v, col0_v, a0_v, sem_i0)
        issue_gathers(row0_v, col0_v, p0_v, q0_v, sem_p0, sem_q0)
        wait_gathers(row0_v, col0_v, p0_v, q0_v, sem_p0, sem_q0)
        compute_and_out(NFULL, row0_v, a0_v, p0_v, q0_v, emb0_v, sem_o0, True)

    plsc.subcore_barrier()

    pltpu.sync_copy(seg_sh.at[pl.ds(off, N_PAD // NS)], zbuf_v)
    pltpu.sync_copy(zbuf_v, seg_hbm.at[cid, pl.ds(off, N_PAD // NS)])


def _sc_edge_pass(p, q, row, col, a128, we2, be2_tiled):
    mesh = plsc.VectorSubcoreMesh(core_axis_name="c", subcore_axis_name="s",
                                  num_cores=NC, num_subcores=NS)
    zeros8 = jnp.zeros((N_PAD, SW), jnp.float32)
    fn = functools.partial(
        pl.kernel,
        out_type=(
            jax.ShapeDtypeStruct((E, F_E), jnp.float32),
            jax.ShapeDtypeStruct((NC, N_PAD, SW), jnp.float32),
        ),
        mesh=mesh,
        compiler_params=pltpu.CompilerParams(needs_layout_passes=False,
                                             use_tc_tiling_on_sc=False),
        scratch_types=[
            pltpu.VMEM((BE,), jnp.int32),
            pltpu.VMEM((BE,), jnp.int32),
            pltpu.VMEM((BE, H), jnp.float32),
            pltpu.VMEM((BE,), jnp.int32),
            pltpu.VMEM((BE,), jnp.int32),
            pltpu.VMEM((BE, H), jnp.float32),
            pltpu.VMEM((BE, H), jnp.float32),
            pltpu.VMEM((BE, H), jnp.float32),
            pltpu.VMEM((BE, H), jnp.float32),
            pltpu.VMEM((BE, H), jnp.float32),
            pltpu.VMEM((BE, F_E), jnp.float32),
            pltpu.VMEM((BE, F_E), jnp.float32),
            pltpu.VMEM((BE, SW), jnp.float32),
            pltpu.VMEM((F_E, H), jnp.float32),
            pltpu.VMEM((16,), jnp.float32),
            pltpu.VMEM((N_PAD // NS, SW), jnp.float32),
            pltpu.VMEM_SHARED((N_PAD, SW), jnp.float32),
            pltpu.SemaphoreType.DMA,
            pltpu.SemaphoreType.DMA,
            pltpu.SemaphoreType.DMA,
            pltpu.SemaphoreType.DMA,
            pltpu.SemaphoreType.DMA,
            pltpu.SemaphoreType.DMA,
            pltpu.SemaphoreType.DMA,
            pltpu.SemaphoreType.DMA,
        ],
    )(_sc_edge_kernel)
    return fn(p, q, row, col, a128, we2, be2_tiled, zeros8)


def _node_mlp_body(x_ref, seg_ref, wn1xt_ref, wn1at_ref, bn1_ref,
                   wn2t_ref, bn2_ref, out_ref):
    tot = seg_ref[0] + seg_ref[1]
    seg = tot[:, 0:F_E]
    cnt = tot[:, F_E:F_E + 1]
    agg = seg / jnp.maximum(cnt, 1.0)
    h2 = (
        jnp.dot(x_ref[...], wn1xt_ref[...], preferred_element_type=jnp.float32)
        + jnp.dot(agg, wn1at_ref[...], preferred_element_type=jnp.float32)
        + bn1_ref[...]
    )
    h2 = jnp.maximum(h2, 0.0)
    out_ref[...] = (
        jnp.dot(h2, wn2t_ref[...], preferred_element_type=jnp.float32)
        + bn2_ref[...]
    )


def _node_mlp(x, seg, wn1xt, wn1at, bn1, wn2t, bn2):
    blk = 2000
    grid = N // blk
    return pl.pallas_call(
        _node_mlp_body,
        grid=(grid,),
        in_specs=[
            pl.BlockSpec((blk, F_N), lambda i: (i, 0)),
            pl.BlockSpec((NC, blk, SW), lambda i: (0, i, 0)),
            pl.BlockSpec((F_N, H), lambda i: (0, 0)),
            pl.BlockSpec((F_E, H), lambda i: (0, 0)),
            pl.BlockSpec((1, H), lambda i: (0, 0)),
            pl.BlockSpec((H, F_N), lambda i: (0, 0)),
            pl.BlockSpec((1, F_N), lambda i: (0, 0)),
        ],
        out_specs=pl.BlockSpec((blk, F_N), lambda i: (i, 0)),
        out_shape=jax.ShapeDtypeStruct((N, F_N), jnp.float32),
    )(x, seg, wn1xt, wn1at, bn1, wn2t, bn2)


def kernel(x, edge_index, edge_attr, We1, be1, We2, be2, Wn1, bn1, Wn2, bn2):
    row = edge_index[0].astype(jnp.int32)
    col = edge_index[1].astype(jnp.int32)

    wst = We1[:, :F_N].T
    wdt = We1[:, F_N:2 * F_N].T
    wat = We1[:, 2 * F_N:].T
    be1_2d = be1.reshape(1, H)
    be2_tiled = jnp.tile(be2, 16 // F_E)
    wn1xt = Wn1[:, :F_N].T
    wn1at = Wn1[:, F_N:].T
    bn1_2d = bn1.reshape(1, H)
    wn2t = Wn2.T
    bn2_2d = bn2.reshape(1, F_N)

    p, q = _node_tables(x, wst, wdt, be1_2d)
    a128 = _a128(edge_attr, wat)
    emb, seg = _sc_edge_pass(p, q, row, col, a128, We2, be2_tiled)
    node_embeddings = _node_mlp(x, seg, wn1xt, wn1at, bn1_2d, wn2t, bn2_2d)
    return (emb, node_embeddings)
